# Initial kernel scaffold; baseline (speedup 1.0000x reference)
import jax
import jax.numpy as jnp
from jax.experimental import pallas as pl


def kernel(columns, logits, adj, proj_w, proj_b, ln_g, ln_b, W, V):
    raise NotImplementedError("write your pallas kernel here")


# speedup vs baseline: 2.0403x; 2.0403x over previous
import functools

import jax
import jax.numpy as jnp
from jax.experimental import pallas as pl
from jax.experimental.pallas import tpu as pltpu


def _fused_rgcn_kernel(cols_ref, log_ref, adj_ref, wc_ref, wl_ref, b_ref,
                       g_ref, bt_ref, wcat_ref, v_ref, out_ref,
                       *, H, L, R, K3, N):
    cols = cols_ref[0]
    z = jnp.dot(cols, wc_ref[...], preferred_element_type=jnp.float32)
    lg = log_ref[0]
    for l in range(L):
        z = z + lg[:, l:l + 1] * wl_ref[l:l + 1, :]
    z = z + b_ref[...]
    inv_h = 1.0 / H
    mu = jnp.sum(z, axis=-1, keepdims=True) * inv_h
    dm = z - mu
    var = jnp.sum(dm * dm, axis=-1, keepdims=True) * inv_h
    xh = dm * jax.lax.rsqrt(var + 1e-5)
    xh = xh * g_ref[...] + bt_ref[...]
    x = 0.5 * xh * (1.0 + jax.lax.erf(xh * 0.7071067811865475))
    xb = x.astype(jnp.bfloat16)

    yb = jnp.dot(xb, wcat_ref[...],
                 preferred_element_type=jnp.float32).astype(jnp.bfloat16)

    adj = adj_ref[0]
    row = jax.lax.broadcasted_iota(jnp.int32, (N, N), 0)
    col = jax.lax.broadcasted_iota(jnp.int32, (N, N), 1)
    eye = (row == col).astype(jnp.float32)
    d = [eye * v_ref[0:1, k:k + 1] for k in range(K3)]
    for r in range(1, R):
        m = (adj == r).astype(jnp.float32)
        for k in range(K3):
            d[k] = d[k] + m * v_ref[r:r + 1, k:k + 1]

    acc = jnp.dot(d[0].astype(jnp.bfloat16), yb[:, 0:H],
                  preferred_element_type=jnp.float32)
    for k in range(1, K3):
        acc = acc + jnp.dot(d[k].astype(jnp.bfloat16), yb[:, k * H:(k + 1) * H],
                            preferred_element_type=jnp.float32)
    out_ref[0] = acc.astype(out_ref.dtype)


def kernel(columns, logits, adj, proj_w, proj_b, ln_g, ln_b, W, V):
    B, N, H = columns.shape
    L = logits.shape[-1]
    R, K3 = V.shape

    wc = proj_w[:, :H].T.astype(jnp.float32)
    wl = jnp.zeros((8, H), jnp.float32).at[:L].set(proj_w[:, H:].T)
    bias = proj_b.reshape(1, H).astype(jnp.float32)
    gamma = ln_g.reshape(1, H).astype(jnp.float32)
    beta = ln_b.reshape(1, H).astype(jnp.float32)
    wcat = W.reshape(H, K3 * H).astype(jnp.bfloat16)
    vp = jnp.zeros((R, 128), jnp.float32).at[:, :K3].set(V)

    flops = 2 * B * N * (H * H + K3 * H * H + K3 * N * H)
    cost = pl.CostEstimate(
        flops=int(flops),
        transcendentals=int(B * N * H),
        bytes_accessed=int(B * N * N * 4 + 2 * B * N * H * 4 + B * N * L * 4),
    )

    out = pl.pallas_call(
        functools.partial(_fused_rgcn_kernel, H=H, L=L, R=R, K3=K3, N=N),
        out_shape=jax.ShapeDtypeStruct((B, N, H), columns.dtype),
        grid=(B,),
        in_specs=[
            pl.BlockSpec((1, N, H), lambda b: (b, 0, 0)),
            pl.BlockSpec((1, N, L), lambda b: (b, 0, 0)),
            pl.BlockSpec((1, N, N), lambda b: (b, 0, 0)),
            pl.BlockSpec((H, H), lambda b: (0, 0)),
            pl.BlockSpec((8, H), lambda b: (0, 0)),
            pl.BlockSpec((1, H), lambda b: (0, 0)),
            pl.BlockSpec((1, H), lambda b: (0, 0)),
            pl.BlockSpec((1, H), lambda b: (0, 0)),
            pl.BlockSpec((H, K3 * H), lambda b: (0, 0)),
            pl.BlockSpec((R, 128), lambda b: (0, 0)),
        ],
        out_specs=pl.BlockSpec((1, N, H), lambda b: (b, 0, 0)),
        compiler_params=pltpu.CompilerParams(
            dimension_semantics=("parallel",)),
        cost_estimate=cost,
    )(columns, logits, adj, wc, wl, bias, gamma, beta, wcat, vp)
    return out


# speedup vs baseline: 2.3143x; 1.1343x over previous
import functools

import jax
import jax.numpy as jnp
from jax.experimental import pallas as pl
from jax.experimental.pallas import tpu as pltpu


def _fused_rgcn_kernel(cols_ref, log_ref, adj_ref, wc_ref, wl_ref, b_ref,
                       g_ref, bt_ref, wall_ref, v_ref, out_ref,
                       *, H, L, R, K3, N):
    cols = cols_ref[0]
    z = jnp.dot(cols, wc_ref[...], preferred_element_type=jnp.float32)
    lg = log_ref[0]
    for l in range(L):
        z = z + lg[:, l:l + 1] * wl_ref[l:l + 1, :]
    z = z + b_ref[...]
    inv_h = 1.0 / H
    mu = jnp.sum(z, axis=-1, keepdims=True) * inv_h
    dm = z - mu
    var = jnp.sum(dm * dm, axis=-1, keepdims=True) * inv_h
    xh = dm * jax.lax.rsqrt(var + 1e-5)
    xh = xh * g_ref[...] + bt_ref[...]
    x = 0.5 * xh * (1.0 + jax.lax.erf(xh * 0.7071067811865475))
    xb = x.astype(jnp.bfloat16)

    yfull = jnp.dot(xb, wall_ref[...],
                    preferred_element_type=jnp.float32)
    acc = yfull[:, 0:H]
    yb = yfull[:, H:].astype(jnp.bfloat16)

    adj = adj_ref[0]
    m = adj == 1
    d = [jnp.where(m, v_ref[1:2, k:k + 1], 0.0) for k in range(K3)]
    for r in range(2, R):
        m = adj == r
        d = [jnp.where(m, v_ref[r:r + 1, k:k + 1], d[k]) for k in range(K3)]

    for k in range(K3):
        acc = acc + jnp.dot(d[k].astype(jnp.bfloat16), yb[:, k * H:(k + 1) * H],
                            preferred_element_type=jnp.float32)
    out_ref[0] = acc.astype(out_ref.dtype)


def kernel(columns, logits, adj, proj_w, proj_b, ln_g, ln_b, W, V):
    B, N, H = columns.shape
    L = logits.shape[-1]
    R, K3 = V.shape

    wc = proj_w[:, :H].T.astype(jnp.float32)
    wl = jnp.zeros((8, H), jnp.float32).at[:L].set(proj_w[:, H:].T)
    bias = proj_b.reshape(1, H).astype(jnp.float32)
    gamma = ln_g.reshape(1, H).astype(jnp.float32)
    beta = ln_b.reshape(1, H).astype(jnp.float32)
    m0 = jnp.einsum("k,ikj->ij", V[0], W)
    wall = jnp.concatenate([m0, W.reshape(H, K3 * H)],
                           axis=1).astype(jnp.bfloat16)
    vp = jnp.zeros((R, 128), jnp.float32).at[:, :K3].set(V)

    flops = 2 * B * N * (H * H + K3 * H * H + K3 * N * H)
    cost = pl.CostEstimate(
        flops=int(flops),
        transcendentals=int(B * N * H),
        bytes_accessed=int(B * N * N * 4 + 2 * B * N * H * 4 + B * N * L * 4),
    )

    out = pl.pallas_call(
        functools.partial(_fused_rgcn_kernel, H=H, L=L, R=R, K3=K3, N=N),
        out_shape=jax.ShapeDtypeStruct((B, N, H), columns.dtype),
        grid=(B,),
        in_specs=[
            pl.BlockSpec((1, N, H), lambda b: (b, 0, 0)),
            pl.BlockSpec((1, N, L), lambda b: (b, 0, 0)),
            pl.BlockSpec((1, N, N), lambda b: (b, 0, 0)),
            pl.BlockSpec((H, H), lambda b: (0, 0)),
            pl.BlockSpec((8, H), lambda b: (0, 0)),
            pl.BlockSpec((1, H), lambda b: (0, 0)),
            pl.BlockSpec((1, H), lambda b: (0, 0)),
            pl.BlockSpec((1, H), lambda b: (0, 0)),
            pl.BlockSpec((H, (K3 + 1) * H), lambda b: (0, 0)),
            pl.BlockSpec((R, 128), lambda b: (0, 0)),
        ],
        out_specs=pl.BlockSpec((1, N, H), lambda b: (b, 0, 0)),
        compiler_params=pltpu.CompilerParams(
            dimension_semantics=("parallel",)),
        cost_estimate=cost,
    )(columns, logits, adj, wc, wl, bias, gamma, beta, wall, vp)
    return out


# speedup vs baseline: 2.3682x; 1.0233x over previous
import functools

import jax
import jax.numpy as jnp
from jax.experimental import pallas as pl
from jax.experimental.pallas import tpu as pltpu


def _fused_rgcn_kernel(cols_ref, log_ref, adj_ref, pw_ref, b_ref,
                       g_ref, bt_ref, w_ref, v_ref, out_ref,
                       *, H, L, R, K3, N):
    cols = cols_ref[0]
    wt = pw_ref[...].T
    z = jnp.dot(cols, wt[:H], preferred_element_type=jnp.float32)
    lg = log_ref[0]
    for l in range(L):
        z = z + lg[:, l:l + 1] * wt[H + l:H + l + 1, :]
    z = z + b_ref[...]
    inv_h = 1.0 / H
    mu = jnp.sum(z, axis=-1, keepdims=True) * inv_h
    dm = z - mu
    var = jnp.sum(dm * dm, axis=-1, keepdims=True) * inv_h
    xh = dm * jax.lax.rsqrt(var + 1e-5)
    xh = xh * g_ref[...] + bt_ref[...]
    x = 0.5 * xh * (1.0 + jax.lax.erf(xh * 0.7071067811865475))
    xb = x.astype(jnp.bfloat16)

    yfull = jnp.dot(xb, w_ref[...].astype(jnp.bfloat16),
                    preferred_element_type=jnp.float32)
    yb = yfull.astype(jnp.bfloat16)

    acc = yfull[:, 0:H] * v_ref[0:1, 0:1]
    for k in range(1, K3):
        acc = acc + yfull[:, k * H:(k + 1) * H] * v_ref[0:1, k:k + 1]

    adj = adj_ref[0]
    m = adj == 1
    d = [jnp.where(m, v_ref[1:2, k:k + 1], 0.0) for k in range(K3)]
    for r in range(2, R):
        m = adj == r
        d = [jnp.where(m, v_ref[r:r + 1, k:k + 1], d[k]) for k in range(K3)]

    for k in range(K3):
        acc = acc + jnp.dot(d[k].astype(jnp.bfloat16), yb[:, k * H:(k + 1) * H],
                            preferred_element_type=jnp.float32)
    out_ref[0] = acc.astype(out_ref.dtype)


def kernel(columns, logits, adj, proj_w, proj_b, ln_g, ln_b, W, V):
    B, N, H = columns.shape
    L = logits.shape[-1]
    R, K3 = V.shape

    bias = proj_b.reshape(1, H)
    gamma = ln_g.reshape(1, H)
    beta = ln_b.reshape(1, H)
    w2d = W.reshape(H, K3 * H)

    flops = 2 * B * N * (H * H + K3 * H * H + K3 * N * H)
    cost = pl.CostEstimate(
        flops=int(flops),
        transcendentals=int(B * N * H),
        bytes_accessed=int(B * N * N * 4 + 2 * B * N * H * 4 + B * N * L * 4),
    )

    out = pl.pallas_call(
        functools.partial(_fused_rgcn_kernel, H=H, L=L, R=R, K3=K3, N=N),
        out_shape=jax.ShapeDtypeStruct((B, N, H), columns.dtype),
        grid=(B,),
        in_specs=[
            pl.BlockSpec((1, N, H), lambda b: (b, 0, 0)),
            pl.BlockSpec((1, N, L), lambda b: (b, 0, 0)),
            pl.BlockSpec((1, N, N), lambda b: (b, 0, 0)),
            pl.BlockSpec((H, H + L), lambda b: (0, 0)),
            pl.BlockSpec((1, H), lambda b: (0, 0)),
            pl.BlockSpec((1, H), lambda b: (0, 0)),
            pl.BlockSpec((1, H), lambda b: (0, 0)),
            pl.BlockSpec((H, K3 * H), lambda b: (0, 0)),
            pl.BlockSpec((R, K3), lambda b: (0, 0)),
        ],
        out_specs=pl.BlockSpec((1, N, H), lambda b: (b, 0, 0)),
        compiler_params=pltpu.CompilerParams(
            dimension_semantics=("parallel",)),
        cost_estimate=cost,
    )(columns, logits, adj, proj_w, bias, gamma, beta, w2d, V)
    return out


# speedup vs baseline: 3.0100x; 1.2710x over previous
import functools

import jax
import jax.numpy as jnp
from jax.experimental import pallas as pl
from jax.experimental.pallas import tpu as pltpu


def _fused_rgcn_kernel(cols_ref, log_ref, adj_ref, pw_ref, b_ref,
                       g_ref, bt_ref, w_ref, v_ref, out_ref,
                       *, H, L, R, K3, N):
    cols = cols_ref[0]
    wt = pw_ref[...].T
    z = jnp.dot(cols, wt[:H], preferred_element_type=jnp.float32)
    lg = log_ref[0]
    for l in range(L):
        z = z + lg[:, l:l + 1] * wt[H + l:H + l + 1, :]
    z = z + b_ref[...]
    inv_h = 1.0 / H
    mu = jnp.sum(z, axis=-1, keepdims=True) * inv_h
    dm = z - mu
    var = jnp.sum(dm * dm, axis=-1, keepdims=True) * inv_h
    xh = dm * jax.lax.rsqrt(var + 1e-5)
    xh = xh * g_ref[...] + bt_ref[...]
    x = 0.5 * xh * (1.0 + jax.lax.erf(xh * 0.7071067811865475))
    xb = x.astype(jnp.bfloat16)

    yfull = jnp.dot(xb, w_ref[...].astype(jnp.bfloat16),
                    preferred_element_type=jnp.float32)
    yb = yfull.astype(jnp.bfloat16)

    acc = yfull[:, 0:H] * v_ref[0:1, 0:1]
    for k in range(1, K3):
        acc = acc + yfull[:, k * H:(k + 1) * H] * v_ref[0:1, k:k + 1]

    adj = adj_ref[0].astype(jnp.int16)
    vb = [[v_ref[r:r + 1, k:k + 1].astype(jnp.bfloat16) for k in range(K3)]
          for r in range(R)]
    zero_b = jnp.zeros((), jnp.bfloat16)
    m = adj == 1
    d = [jnp.where(m, vb[1][k], zero_b) for k in range(K3)]
    for r in range(2, R):
        m = adj == r
        d = [jnp.where(m, vb[r][k], d[k]) for k in range(K3)]

    for k in range(K3):
        acc = acc + jnp.dot(d[k], yb[:, k * H:(k + 1) * H],
                            preferred_element_type=jnp.float32)
    out_ref[0] = acc.astype(out_ref.dtype)


def kernel(columns, logits, adj, proj_w, proj_b, ln_g, ln_b, W, V):
    B, N, H = columns.shape
    L = logits.shape[-1]
    R, K3 = V.shape

    bias = proj_b.reshape(1, H)
    gamma = ln_g.reshape(1, H)
    beta = ln_b.reshape(1, H)
    w2d = W.reshape(H, K3 * H)

    flops = 2 * B * N * (H * H + K3 * H * H + K3 * N * H)
    cost = pl.CostEstimate(
        flops=int(flops),
        transcendentals=int(B * N * H),
        bytes_accessed=int(B * N * N * 4 + 2 * B * N * H * 4 + B * N * L * 4),
    )

    out = pl.pallas_call(
        functools.partial(_fused_rgcn_kernel, H=H, L=L, R=R, K3=K3, N=N),
        out_shape=jax.ShapeDtypeStruct((B, N, H), columns.dtype),
        grid=(B,),
        in_specs=[
            pl.BlockSpec((1, N, H), lambda b: (b, 0, 0)),
            pl.BlockSpec((1, N, L), lambda b: (b, 0, 0)),
            pl.BlockSpec((1, N, N), lambda b: (b, 0, 0)),
            pl.BlockSpec((H, H + L), lambda b: (0, 0)),
            pl.BlockSpec((1, H), lambda b: (0, 0)),
            pl.BlockSpec((1, H), lambda b: (0, 0)),
            pl.BlockSpec((1, H), lambda b: (0, 0)),
            pl.BlockSpec((H, K3 * H), lambda b: (0, 0)),
            pl.BlockSpec((R, K3), lambda b: (0, 0)),
        ],
        out_specs=pl.BlockSpec((1, N, H), lambda b: (b, 0, 0)),
        compiler_params=pltpu.CompilerParams(
            dimension_semantics=("parallel",)),
        cost_estimate=cost,
    )(columns, logits, adj, proj_w, bias, gamma, beta, w2d, V)
    return out


# speedup vs baseline: 3.6664x; 1.2181x over previous
import functools

import jax
import jax.numpy as jnp
from jax.experimental import pallas as pl
from jax.experimental.pallas import tpu as pltpu


def _fused_rgcn_kernel(cols_ref, log_ref, adj_ref, pw_ref, b_ref,
                       g_ref, bt_ref, w_ref, v_ref, out_ref,
                       *, H, L, R, K3, N, BE):
    wb = w_ref[...].astype(jnp.bfloat16)
    vb = [[v_ref[r:r + 1, k:k + 1].astype(jnp.bfloat16) for k in range(K3)]
          for r in range(R)]
    zero_b = jnp.zeros((), jnp.bfloat16)
    cdims = (((1,), (1,)), ((), ()))

    for e in range(BE):
        cols = cols_ref[e]
        z = jax.lax.dot_general(cols, pw_ref[:, :H], cdims,
                                preferred_element_type=jnp.float32)
        z = z + jax.lax.dot_general(log_ref[e], pw_ref[:, H:], cdims,
                                    preferred_element_type=jnp.float32)
        z = z + b_ref[...]
        inv_h = 1.0 / H
        mu = jnp.sum(z, axis=-1, keepdims=True) * inv_h
        dm = z - mu
        var = jnp.sum(dm * dm, axis=-1, keepdims=True) * inv_h
        xh = dm * jax.lax.rsqrt(var + 1e-5)
        xh = xh * g_ref[...] + bt_ref[...]
        x = 0.5 * xh * (1.0 + jax.lax.erf(xh * 0.7071067811865475))
        xb = x.astype(jnp.bfloat16)

        yfull = jnp.dot(xb, wb, preferred_element_type=jnp.float32)
        yb = yfull.astype(jnp.bfloat16)

        acc = yfull[:, 0:H] * v_ref[0:1, 0:1]
        for k in range(1, K3):
            acc = acc + yfull[:, k * H:(k + 1) * H] * v_ref[0:1, k:k + 1]

        adj = adj_ref[e].astype(jnp.int16)
        m = adj == 1
        d = [jnp.where(m, vb[1][k], zero_b) for k in range(K3)]
        for r in range(2, R):
            m = adj == r
            d = [jnp.where(m, vb[r][k], d[k]) for k in range(K3)]

        for k in range(K3):
            acc = acc + jnp.dot(d[k], yb[:, k * H:(k + 1) * H],
                                preferred_element_type=jnp.float32)
        out_ref[e] = acc.astype(out_ref.dtype)


def kernel(columns, logits, adj, proj_w, proj_b, ln_g, ln_b, W, V):
    B, N, H = columns.shape
    L = logits.shape[-1]
    R, K3 = V.shape

    bias = proj_b.reshape(1, H)
    gamma = ln_g.reshape(1, H)
    beta = ln_b.reshape(1, H)
    w2d = W.reshape(H, K3 * H)

    flops = 2 * B * N * (H * H + K3 * H * H + K3 * N * H)
    cost = pl.CostEstimate(
        flops=int(flops),
        transcendentals=int(B * N * H),
        bytes_accessed=int(B * N * N * 4 + 2 * B * N * H * 4 + B * N * L * 4),
    )

    BE = 2 if B % 2 == 0 else 1
    out = pl.pallas_call(
        functools.partial(_fused_rgcn_kernel, H=H, L=L, R=R, K3=K3, N=N, BE=BE),
        out_shape=jax.ShapeDtypeStruct((B, N, H), columns.dtype),
        grid=(B // BE,),
        in_specs=[
            pl.BlockSpec((BE, N, H), lambda b: (b, 0, 0)),
            pl.BlockSpec((BE, N, L), lambda b: (b, 0, 0)),
            pl.BlockSpec((BE, N, N), lambda b: (b, 0, 0)),
            pl.BlockSpec((H, H + L), lambda b: (0, 0)),
            pl.BlockSpec((1, H), lambda b: (0, 0)),
            pl.BlockSpec((1, H), lambda b: (0, 0)),
            pl.BlockSpec((1, H), lambda b: (0, 0)),
            pl.BlockSpec((H, K3 * H), lambda b: (0, 0)),
            pl.BlockSpec((R, K3), lambda b: (0, 0)),
        ],
        out_specs=pl.BlockSpec((BE, N, H), lambda b: (b, 0, 0)),
        compiler_params=pltpu.CompilerParams(
            dimension_semantics=("parallel",)),
        cost_estimate=cost,
    )(columns, logits, adj, proj_w, bias, gamma, beta, w2d, V)
    return out


# speedup vs baseline: 3.7799x; 1.0310x over previous
import functools

import jax
import jax.numpy as jnp
from jax.experimental import pallas as pl
from jax.experimental.pallas import tpu as pltpu


def _fused_rgcn_kernel(cols_ref, log_ref, adj_ref, pw_ref, b_ref,
                       g_ref, bt_ref, w_ref, v_ref, out_ref,
                       *, H, L, R, K3, N, BE):
    wb = w_ref[...].astype(jnp.bfloat16)
    vb = [[v_ref[r:r + 1, k:k + 1].astype(jnp.bfloat16) for k in range(K3)]
          for r in range(R)]
    zero_b = jnp.zeros((), jnp.bfloat16)
    cdims = (((1,), (1,)), ((), ()))

    for e in range(BE):
        cols = cols_ref[e]
        z = jax.lax.dot_general(cols, pw_ref[:, :H], cdims,
                                preferred_element_type=jnp.float32)
        z = z + jax.lax.dot_general(log_ref[e], pw_ref[:, H:], cdims,
                                    preferred_element_type=jnp.float32)
        z = z + b_ref[...]
        inv_h = 1.0 / H
        mu = jnp.sum(z, axis=-1, keepdims=True) * inv_h
        dm = z - mu
        var = jnp.sum(dm * dm, axis=-1, keepdims=True) * inv_h
        xh = dm * jax.lax.rsqrt(var + 1e-5)
        xh = xh * g_ref[...] + bt_ref[...]
        x = 0.5 * xh * (1.0 + jax.lax.erf(xh * 0.7071067811865475))
        xb = x.astype(jnp.bfloat16)

        yfull = jnp.dot(xb, wb, preferred_element_type=jnp.float32)
        yb = yfull.astype(jnp.bfloat16)

        acc = yfull[:, 0:H] * v_ref[0:1, 0:1]
        for k in range(1, K3):
            acc = acc + yfull[:, k * H:(k + 1) * H] * v_ref[0:1, k:k + 1]

        adj = adj_ref[e].astype(jnp.int16)
        m = adj == 1
        d = [jnp.where(m, vb[1][k], zero_b) for k in range(K3)]
        for r in range(2, R):
            m = adj == r
            d = [jnp.where(m, vb[r][k], d[k]) for k in range(K3)]

        for k in range(K3):
            acc = acc + jnp.dot(d[k], yb[:, k * H:(k + 1) * H],
                                preferred_element_type=jnp.float32)
        out_ref[e] = acc.astype(out_ref.dtype)


def kernel(columns, logits, adj, proj_w, proj_b, ln_g, ln_b, W, V):
    B, N, H = columns.shape
    L = logits.shape[-1]
    R, K3 = V.shape

    bias = proj_b.reshape(1, H)
    gamma = ln_g.reshape(1, H)
    beta = ln_b.reshape(1, H)
    w2d = W.reshape(H, K3 * H)

    flops = 2 * B * N * (H * H + K3 * H * H + K3 * N * H)
    cost = pl.CostEstimate(
        flops=int(flops),
        transcendentals=int(B * N * H),
        bytes_accessed=int(B * N * N * 4 + 2 * B * N * H * 4 + B * N * L * 4),
    )

    BE = 4 if B % 4 == 0 else (2 if B % 2 == 0 else 1)
    out = pl.pallas_call(
        functools.partial(_fused_rgcn_kernel, H=H, L=L, R=R, K3=K3, N=N, BE=BE),
        out_shape=jax.ShapeDtypeStruct((B, N, H), columns.dtype),
        grid=(B // BE,),
        in_specs=[
            pl.BlockSpec((BE, N, H), lambda b: (b, 0, 0)),
            pl.BlockSpec((BE, N, L), lambda b: (b, 0, 0)),
            pl.BlockSpec((BE, N, N), lambda b: (b, 0, 0)),
            pl.BlockSpec((H, H + L), lambda b: (0, 0)),
            pl.BlockSpec((1, H), lambda b: (0, 0)),
            pl.BlockSpec((1, H), lambda b: (0, 0)),
            pl.BlockSpec((1, H), lambda b: (0, 0)),
            pl.BlockSpec((H, K3 * H), lambda b: (0, 0)),
            pl.BlockSpec((R, K3), lambda b: (0, 0)),
        ],
        out_specs=pl.BlockSpec((BE, N, H), lambda b: (b, 0, 0)),
        compiler_params=pltpu.CompilerParams(
            dimension_semantics=("parallel",)),
        cost_estimate=cost,
    )(columns, logits, adj, proj_w, bias, gamma, beta, w2d, V)
    return out


# speedup vs baseline: 4.0170x; 1.0627x over previous
import functools

import jax
import jax.numpy as jnp
from jax.experimental import pallas as pl
from jax.experimental.pallas import tpu as pltpu


def _fused_rgcn_kernel(cols_ref, log_ref, adj_ref, pw_ref, b_ref,
                       g_ref, bt_ref, w_ref, v_ref, out_ref,
                       *, H, L, R, K3, N, BE):
    wb = w_ref[...].astype(jnp.bfloat16)
    vb = [[v_ref[r:r + 1, k:k + 1].astype(jnp.bfloat16) for k in range(K3)]
          for r in range(R)]
    zero_b = jnp.zeros((), jnp.bfloat16)
    cdims = (((1,), (1,)), ((), ()))

    for e in range(BE):
        cols = cols_ref[e]
        z = jax.lax.dot_general(cols, pw_ref[:, :H], cdims,
                                preferred_element_type=jnp.float32)
        z = z + jax.lax.dot_general(log_ref[e], pw_ref[:, H:], cdims,
                                    preferred_element_type=jnp.float32)
        z = z + b_ref[...]
        inv_h = 1.0 / H
        mu = jnp.sum(z, axis=-1, keepdims=True) * inv_h
        dm = z - mu
        var = jnp.sum(dm * dm, axis=-1, keepdims=True) * inv_h
        xh = dm * jax.lax.rsqrt(var + 1e-5)
        xh = xh * g_ref[...] + bt_ref[...]
        x = 0.5 * xh * (1.0 + jax.lax.erf(xh * 0.7071067811865475))
        xb = x.astype(jnp.bfloat16)

        yfull = jnp.dot(xb, wb, preferred_element_type=jnp.float32)
        yb = yfull.astype(jnp.bfloat16)

        acc = yfull[:, 0:H] * v_ref[0:1, 0:1]
        for k in range(1, K3):
            acc = acc + yfull[:, k * H:(k + 1) * H] * v_ref[0:1, k:k + 1]

        adj = adj_ref[e]
        d = []
        for k in range(K3):
            tbl = jnp.where(jax.lax.broadcasted_iota(jnp.int32, (R, 1), 0) == 0,
                            0.0, v_ref[:, k:k + 1])
            tbl_bc = jnp.broadcast_to(tbl, (R, N))
            d.append(jnp.take_along_axis(tbl_bc, adj, axis=0)
                     .astype(jnp.bfloat16))

        for k in range(K3):
            acc = acc + jnp.dot(d[k], yb[:, k * H:(k + 1) * H],
                                preferred_element_type=jnp.float32)
        out_ref[e] = acc.astype(out_ref.dtype)


def kernel(columns, logits, adj, proj_w, proj_b, ln_g, ln_b, W, V):
    B, N, H = columns.shape
    L = logits.shape[-1]
    R, K3 = V.shape

    bias = proj_b.reshape(1, H)
    gamma = ln_g.reshape(1, H)
    beta = ln_b.reshape(1, H)
    w2d = W.reshape(H, K3 * H)

    flops = 2 * B * N * (H * H + K3 * H * H + K3 * N * H)
    cost = pl.CostEstimate(
        flops=int(flops),
        transcendentals=int(B * N * H),
        bytes_accessed=int(B * N * N * 4 + 2 * B * N * H * 4 + B * N * L * 4),
    )

    BE = 4 if B % 4 == 0 else (2 if B % 2 == 0 else 1)
    out = pl.pallas_call(
        functools.partial(_fused_rgcn_kernel, H=H, L=L, R=R, K3=K3, N=N, BE=BE),
        out_shape=jax.ShapeDtypeStruct((B, N, H), columns.dtype),
        grid=(B // BE,),
        in_specs=[
            pl.BlockSpec((BE, N, H), lambda b: (b, 0, 0)),
            pl.BlockSpec((BE, N, L), lambda b: (b, 0, 0)),
            pl.BlockSpec((BE, N, N), lambda b: (b, 0, 0)),
            pl.BlockSpec((H, H + L), lambda b: (0, 0)),
            pl.BlockSpec((1, H), lambda b: (0, 0)),
            pl.BlockSpec((1, H), lambda b: (0, 0)),
            pl.BlockSpec((1, H), lambda b: (0, 0)),
            pl.BlockSpec((H, K3 * H), lambda b: (0, 0)),
            pl.BlockSpec((R, K3), lambda b: (0, 0)),
        ],
        out_specs=pl.BlockSpec((BE, N, H), lambda b: (b, 0, 0)),
        compiler_params=pltpu.CompilerParams(
            dimension_semantics=("parallel",)),
        cost_estimate=cost,
    )(columns, logits, adj, proj_w, bias, gamma, beta, w2d, V)
    return out


# speedup vs baseline: 4.0372x; 1.0050x over previous
import functools

import jax
import jax.numpy as jnp
from jax.experimental import pallas as pl
from jax.experimental.pallas import tpu as pltpu


def _fused_rgcn_kernel(cols_ref, log_ref, adj_ref, pw_ref, b_ref,
                       g_ref, bt_ref, w_ref, v_ref, out_ref,
                       *, H, L, R, K3, N, BE):
    wb = w_ref[...].astype(jnp.bfloat16)
    m0 = wb[:, 0:H] * v_ref[0:1, 0:1].astype(jnp.bfloat16)
    for k in range(1, K3):
        m0 = m0 + wb[:, k * H:(k + 1) * H] * v_ref[0:1, k:k + 1].astype(jnp.bfloat16)
    wbig = jnp.concatenate([m0, wb], axis=1)
    cdims = (((1,), (1,)), ((), ()))

    for e in range(BE):
        cols = cols_ref[e]
        z = jax.lax.dot_general(cols, pw_ref[:, :H], cdims,
                                preferred_element_type=jnp.float32)
        z = z + jax.lax.dot_general(log_ref[e], pw_ref[:, H:], cdims,
                                    preferred_element_type=jnp.float32)
        z = z + b_ref[...]
        inv_h = 1.0 / H
        mu = jnp.sum(z, axis=-1, keepdims=True) * inv_h
        dm = z - mu
        var = jnp.sum(dm * dm, axis=-1, keepdims=True) * inv_h
        xh = dm * jax.lax.rsqrt(var + 1e-5)
        xh = xh * g_ref[...] + bt_ref[...]
        x = 0.5 * xh * (1.0 + jax.lax.erf(xh * 0.7071067811865475))
        xb = x.astype(jnp.bfloat16)

        yfull = jnp.dot(xb, wbig, preferred_element_type=jnp.float32)
        acc = yfull[:, 0:H]
        yb = yfull[:, H:].astype(jnp.bfloat16)

        adj = adj_ref[e]
        d = []
        for k in range(K3):
            tbl = jnp.where(jax.lax.broadcasted_iota(jnp.int32, (R, 1), 0) == 0,
                            0.0, v_ref[:, k:k + 1])
            tbl_bc = jnp.broadcast_to(tbl, (R, N))
            d.append(jnp.take_along_axis(tbl_bc, adj, axis=0)
                     .astype(jnp.bfloat16))

        for k in range(K3):
            acc = acc + jnp.dot(d[k], yb[:, k * H:(k + 1) * H],
                                preferred_element_type=jnp.float32)
        out_ref[e] = acc.astype(out_ref.dtype)


def kernel(columns, logits, adj, proj_w, proj_b, ln_g, ln_b, W, V):
    B, N, H = columns.shape
    L = logits.shape[-1]
    R, K3 = V.shape

    bias = proj_b.reshape(1, H)
    gamma = ln_g.reshape(1, H)
    beta = ln_b.reshape(1, H)
    w2d = W.reshape(H, K3 * H)

    flops = 2 * B * N * (H * H + K3 * H * H + K3 * N * H)
    cost = pl.CostEstimate(
        flops=int(flops),
        transcendentals=int(B * N * H),
        bytes_accessed=int(B * N * N * 4 + 2 * B * N * H * 4 + B * N * L * 4),
    )

    BE = 4 if B % 4 == 0 else (2 if B % 2 == 0 else 1)
    out = pl.pallas_call(
        functools.partial(_fused_rgcn_kernel, H=H, L=L, R=R, K3=K3, N=N, BE=BE),
        out_shape=jax.ShapeDtypeStruct((B, N, H), columns.dtype),
        grid=(B // BE,),
        in_specs=[
            pl.BlockSpec((BE, N, H), lambda b: (b, 0, 0)),
            pl.BlockSpec((BE, N, L), lambda b: (b, 0, 0)),
            pl.BlockSpec((BE, N, N), lambda b: (b, 0, 0)),
            pl.BlockSpec((H, H + L), lambda b: (0, 0)),
            pl.BlockSpec((1, H), lambda b: (0, 0)),
            pl.BlockSpec((1, H), lambda b: (0, 0)),
            pl.BlockSpec((1, H), lambda b: (0, 0)),
            pl.BlockSpec((H, K3 * H), lambda b: (0, 0)),
            pl.BlockSpec((R, K3), lambda b: (0, 0)),
        ],
        out_specs=pl.BlockSpec((BE, N, H), lambda b: (b, 0, 0)),
        compiler_params=pltpu.CompilerParams(
            dimension_semantics=("parallel",)),
        cost_estimate=cost,
    )(columns, logits, adj, proj_w, bias, gamma, beta, w2d, V)
    return out
